# Initial kernel scaffold; baseline (speedup 1.0000x reference)
import functools

import jax
import jax.numpy as jnp
from jax import lax
from jax.experimental import pallas as pl
from jax.experimental.pallas import tpu as pltpu
from jax.experimental.pallas import tpu_sc as plsc

B = 16384
L = 20
EMB = 32
NC = 2
NS = 16
NW = NC * NS
BPW = B // NW
SUB = 128
NSUB = BPW // SUB

_mesh = plsc.VectorSubcoreMesh(
    core_axis_name="c", subcore_axis_name="s", num_cores=NC, num_subcores=NS
)


@functools.partial(
    pl.kernel,
    out_type=jax.ShapeDtypeStruct((B, 3 * EMB), jnp.float32),
    mesh=_mesh,
    scratch_types=[
        pltpu.VMEM((L, NSUB, SUB), jnp.int32),
        pltpu.VMEM((NSUB, SUB), jnp.int32),
        pltpu.VMEM((NSUB, SUB), jnp.int32),
        pltpu.VMEM((BPW, EMB), jnp.float32),
        pltpu.VMEM((BPW, EMB), jnp.float32),
        pltpu.VMEM((BPW, EMB), jnp.float32),
        pltpu.VMEM((BPW,), jnp.float32),
        pltpu.VMEM((BPW,), jnp.float32),
        pltpu.VMEM((EMB,), jnp.float32),
        pltpu.SemaphoreType.DMA,
        pltpu.SemaphoreType.DMA,
    ],
)
def _sc_embed(
    gidx_hbm, tidx_hbm, aidx_hbm, gtab, ttab, atab, out_hbm,
    gid_v, tid_v, aid_v, acc_v, t_v, a_v, n0_v, rec_v, row0_v,
    gsem, tsem,
):
    wid = lax.axis_index("c") * NS + lax.axis_index("s")
    base = wid * BPW

    pltpu.sync_copy(gidx_hbm.at[:, pl.ds(wid * NSUB, NSUB), :], gid_v)
    pltpu.sync_copy(tidx_hbm.at[pl.ds(wid * NSUB, NSUB), :], tid_v)
    pltpu.sync_copy(aidx_hbm.at[pl.ds(wid * NSUB, NSUB), :], aid_v)
    pltpu.sync_copy(gtab.at[0], row0_v)

    for sub in range(NSUB):
        pltpu.async_copy(
            gtab.at[gid_v.at[0, sub]], acc_v.at[pl.ds(sub * SUB, SUB)], gsem
        )
    for sub in range(NSUB):
        pltpu.async_copy(
            ttab.at[tid_v.at[sub]], t_v.at[pl.ds(sub * SUB, SUB)], tsem
        )
        pltpu.async_copy(
            atab.at[aid_v.at[sub]], a_v.at[pl.ds(sub * SUB, SUB)], tsem
        )
    for sub in range(NSUB):
        pltpu.make_async_copy(
            gtab.at[gid_v.at[0, sub]], acc_v.at[pl.ds(sub * SUB, SUB)], gsem
        ).wait()

    def fire(i, _):
        l = 1 + i // NSUB
        sub = i % NSUB
        pltpu.async_copy(
            gtab.at[gid_v.at[l, sub]],
            acc_v.at[pl.ds(sub * SUB, SUB)],
            gsem,
            add=True,
        )
        return 0

    lax.fori_loop(0, (L - 1) * NSUB, fire, 0)

    def count_body(i, _):
        sub = i // (SUB // 16)
        off = (i % (SUB // 16)) * 16
        acc = jnp.zeros((16,), jnp.float32)
        for l in range(L):
            ids = gid_v[l, sub, pl.ds(off, 16)]
            acc = acc + jnp.where(ids == 0, 1.0, 0.0).astype(jnp.float32)
        n0_v[pl.ds(sub * SUB + off, 16)] = acc
        cnt = jnp.float32(L) - acc
        rec_v[pl.ds(sub * SUB + off, 16)] = jnp.where(
            cnt > 0.5, jnp.float32(1.0) / cnt, jnp.float32(0.0)
        )
        return 0

    lax.fori_loop(0, BPW // 16, count_body, 0)

    def drain(i, _):
        pltpu.make_async_copy(
            gtab.at[gid_v.at[0, 0]], acc_v.at[pl.ds(0, SUB)], gsem
        ).wait()
        return 0

    lax.fori_loop(0, (L - 1) * NSUB, drain, 0)

    r0a = row0_v[pl.ds(0, 16)]
    r0b = row0_v[pl.ds(16, 16)]

    def norm(r, _):
        n0 = n0_v[r]
        rec = rec_v[r]
        v0 = acc_v[r, pl.ds(0, 16)]
        v1 = acc_v[r, pl.ds(16, 16)]
        acc_v[r, pl.ds(0, 16)] = (v0 - n0 * r0a) * rec
        acc_v[r, pl.ds(16, 16)] = (v1 - n0 * r0b) * rec
        return 0

    lax.fori_loop(0, BPW, norm, 0)

    for _ in range(2 * NSUB):
        pltpu.make_async_copy(
            ttab.at[tid_v.at[0]], t_v.at[pl.ds(0, SUB)], tsem
        ).wait()

    pltpu.sync_copy(acc_v, out_hbm.at[pl.ds(base, BPW), pl.ds(0, EMB)])
    pltpu.sync_copy(t_v, out_hbm.at[pl.ds(base, BPW), pl.ds(EMB, EMB)])
    pltpu.sync_copy(a_v, out_hbm.at[pl.ds(base, BPW), pl.ds(2 * EMB, EMB)])


def kernel(genre_ids, type_ids, audience_ids, genre_table, type_table,
           audience_table):
    gids = genre_ids.astype(jnp.int32)
    tids = type_ids.astype(jnp.int32)
    aids = audience_ids.astype(jnp.int32)
    gidx = gids.T.reshape(L, B // SUB, SUB)
    tidx = tids.reshape(B // SUB, SUB)
    aidx = aids.reshape(B // SUB, SUB)
    return _sc_embed(gidx, tidx, aidx, genre_table, type_table,
                     audience_table)


# speedup vs baseline: 18.8217x; 18.8217x over previous
import functools

import jax
import jax.numpy as jnp
from jax import lax
from jax.experimental import pallas as pl
from jax.experimental.pallas import tpu as pltpu
from jax.experimental.pallas import tpu_sc as plsc

B = 16384
L = 20
EMB = 32
NC = 2
NS = 16
NW = NC * NS
BPW = B // NW
SUB = 128
NSUB = BPW // SUB

_mesh = plsc.VectorSubcoreMesh(
    core_axis_name="c", subcore_axis_name="s", num_cores=NC, num_subcores=NS
)


@functools.partial(
    pl.kernel,
    out_type=jax.ShapeDtypeStruct((B, 3 * EMB), jnp.float32),
    mesh=_mesh,
    compiler_params=pltpu.CompilerParams(use_tc_tiling_on_sc=False),
    scratch_types=[
        pltpu.VMEM((L, NSUB, SUB), jnp.int32),
        pltpu.VMEM((NSUB, SUB), jnp.int32),
        pltpu.VMEM((NSUB, SUB), jnp.int32),
        pltpu.VMEM((BPW, EMB), jnp.float32),
        pltpu.VMEM((BPW, EMB), jnp.float32),
        pltpu.VMEM((BPW, EMB), jnp.float32),
        pltpu.VMEM((BPW,), jnp.float32),
        pltpu.VMEM((BPW,), jnp.float32),
        pltpu.VMEM((EMB,), jnp.float32),
        pltpu.SemaphoreType.DMA,
        pltpu.SemaphoreType.DMA,
    ],
)
def _sc_embed(
    gidx_hbm, tidx_hbm, aidx_hbm, gtab, ttab, atab, out_hbm,
    gid_v, tid_v, aid_v, acc_v, t_v, a_v, n0_v, rec_v, row0_v,
    gsem, tsem,
):
    wid = lax.axis_index("c") * NS + lax.axis_index("s")
    base = wid * BPW

    pltpu.sync_copy(gidx_hbm.at[:, pl.ds(wid * NSUB, NSUB), :], gid_v)
    pltpu.sync_copy(tidx_hbm.at[pl.ds(wid * NSUB, NSUB), :], tid_v)
    pltpu.sync_copy(aidx_hbm.at[pl.ds(wid * NSUB, NSUB), :], aid_v)
    pltpu.sync_copy(gtab.at[0], row0_v)

    for sub in range(NSUB):
        pltpu.async_copy(
            gtab.at[gid_v.at[0, sub]], acc_v.at[pl.ds(sub * SUB, SUB)], gsem
        )
    for sub in range(NSUB):
        pltpu.async_copy(
            ttab.at[tid_v.at[sub]], t_v.at[pl.ds(sub * SUB, SUB)], tsem
        )
        pltpu.async_copy(
            atab.at[aid_v.at[sub]], a_v.at[pl.ds(sub * SUB, SUB)], tsem
        )
    for sub in range(NSUB):
        pltpu.make_async_copy(
            gtab.at[gid_v.at[0, sub]], acc_v.at[pl.ds(sub * SUB, SUB)], gsem
        ).wait()

    def fire(i, _):
        l = 1 + i // NSUB
        sub = i % NSUB
        pltpu.async_copy(
            gtab.at[gid_v.at[l, sub]],
            acc_v.at[pl.ds(sub * SUB, SUB)],
            gsem,
            add=True,
        )
        return 0

    lax.fori_loop(0, (L - 1) * NSUB, fire, 0)

    def count_body(i, _):
        sub = i // (SUB // 16)
        off = (i % (SUB // 16)) * 16
        acc = jnp.zeros((16,), jnp.float32)
        for l in range(L):
            ids = gid_v[l, sub, pl.ds(off, 16)]
            acc = acc + jnp.where(ids == 0, 1.0, 0.0).astype(jnp.float32)
        n0_v[pl.ds(sub * SUB + off, 16)] = acc
        cnt = jnp.float32(L) - acc
        rec_v[pl.ds(sub * SUB + off, 16)] = jnp.where(
            cnt > 0.5, jnp.float32(1.0) / cnt, jnp.float32(0.0)
        )
        return 0

    lax.fori_loop(0, BPW // 16, count_body, 0)

    def drain(i, _):
        pltpu.make_async_copy(
            gtab.at[gid_v.at[0, 0]], acc_v.at[pl.ds(0, SUB)], gsem
        ).wait()
        return 0

    lax.fori_loop(0, (L - 1) * NSUB, drain, 0)

    r0a = row0_v[pl.ds(0, 16)]
    r0b = row0_v[pl.ds(16, 16)]

    def norm(g, _):
        n0g = n0_v[pl.ds(g * 16, 16)]
        recg = rec_v[pl.ds(g * 16, 16)]
        for j in range(16):
            r = g * 16 + j
            n0 = n0g[j]
            rec = recg[j]
            v0 = acc_v[r, pl.ds(0, 16)]
            v1 = acc_v[r, pl.ds(16, 16)]
            acc_v[r, pl.ds(0, 16)] = (v0 - n0 * r0a) * rec
            acc_v[r, pl.ds(16, 16)] = (v1 - n0 * r0b) * rec
        return 0

    lax.fori_loop(0, BPW // 16, norm, 0)

    for _ in range(2 * NSUB):
        pltpu.make_async_copy(
            ttab.at[tid_v.at[0]], t_v.at[pl.ds(0, SUB)], tsem
        ).wait()

    pltpu.sync_copy(acc_v, out_hbm.at[pl.ds(base, BPW), pl.ds(0, EMB)])
    pltpu.sync_copy(t_v, out_hbm.at[pl.ds(base, BPW), pl.ds(EMB, EMB)])
    pltpu.sync_copy(a_v, out_hbm.at[pl.ds(base, BPW), pl.ds(2 * EMB, EMB)])


def kernel(genre_ids, type_ids, audience_ids, genre_table, type_table,
           audience_table):
    gids = genre_ids.astype(jnp.int32)
    tids = type_ids.astype(jnp.int32)
    aids = audience_ids.astype(jnp.int32)
    gidx = gids.T.reshape(L, B // SUB, SUB)
    tidx = tids.reshape(B // SUB, SUB)
    aidx = aids.reshape(B // SUB, SUB)
    return _sc_embed(gidx, tidx, aidx, genre_table, type_table,
                     audience_table)
